# baseline (speedup 1.0000x reference)
import functools

import jax
import jax.numpy as jnp
from jax import lax
from jax.experimental import pallas as pl
from jax.experimental.pallas import tpu as pltpu
from jax.experimental.pallas import tpu_sc as plsc

B = 4096
D = 64
L = 5
NC = 2
NS = 16
NW = NC * NS
BPW = B // NW
LANES = 16


def _body(uid_hbm, iid_hbm, ue_hbm, ie_hbm, ib_hbm, ub_hbm, out_hbm,
          uid_v, iid_v, ibq_v, ubq_v, u_v, v_v, ib_v, ub_v, out_v, sem):
    wid = lax.axis_index("s") * NC + lax.axis_index("c")
    base = wid * BPW

    pltpu.sync_copy(uid_hbm.at[pl.ds(base, BPW)], uid_v)
    pltpu.sync_copy(iid_hbm.at[pl.ds(base, BPW)], iid_v)

    for c in range(BPW // LANES):
        sl = pl.ds(c * LANES, LANES)
        ibq_v[sl] = lax.shift_right_logical(iid_v[sl], 4)
        ubq_v[sl] = lax.shift_right_logical(uid_v[sl], 2)

    cp_u = pltpu.async_copy(ue_hbm.at[uid_v], u_v, sem)
    cp_v = pltpu.async_copy(ie_hbm.at[iid_v], v_v, sem)
    cp_ib = pltpu.async_copy(ib_hbm.at[ibq_v], ib_v, sem)
    cp_ub = pltpu.async_copy(ub_hbm.at[ubq_v], ub_v, sem)
    cp_u.wait()
    cp_v.wait()
    cp_ib.wait()
    cp_ub.wait()

    lane = lax.iota(jnp.int32, LANES)
    one = jnp.float32(1.0)

    def _exp(x):
        t = x * jnp.float32(1.4426950408889634)
        n = t.astype(jnp.int32)
        g = (t - n.astype(jnp.float32)) * jnp.float32(0.6931471805599453)
        p = one + g * jnp.float32(1 / 7.0)
        for r in (6.0, 5.0, 4.0, 3.0, 2.0, 1.0):
            p = one + g * jnp.float32(1 / r) * p
        scale = lax.bitcast_convert_type(
            lax.shift_left(n + 127, jnp.full((LANES,), 23, jnp.int32)),
            jnp.float32)
        return p * scale

    for c in range(BPW // LANES):
        sl = pl.ds(c * LANES, LANES)
        rows = c * LANES + lane

        def dot_step(d, acc):
            col = jnp.full((LANES,), d, jnp.int32)
            ug = plsc.load_gather(u_v, [rows, col])
            vg = plsc.load_gather(v_v, [rows, col])
            return acc + ug * vg

        y = lax.fori_loop(0, D, dot_step, jnp.zeros((LANES,), jnp.float32))
        ib_col = jnp.bitwise_and(iid_v[sl], 15)
        y = y + plsc.load_gather(ib_v, [rows, ib_col])

        ub_col = lax.shift_left(jnp.bitwise_and(uid_v[sl], 3),
                                jnp.full((LANES,), 2, jnp.int32))
        cut = plsc.load_gather(ub_v, [rows, ub_col])
        d_prev = one / (one + _exp(y - cut))
        plsc.store_scatter(out_v, [rows, jnp.zeros((LANES,), jnp.int32)],
                           d_prev)
        for k in range(1, L - 1):
            cut = cut + _exp(plsc.load_gather(ub_v, [rows, ub_col + k]))
            d_k = one / (one + _exp(y - cut))
            plsc.store_scatter(
                out_v, [rows, jnp.full((LANES,), k, jnp.int32)], d_k - d_prev)
            d_prev = d_k
        plsc.store_scatter(
            out_v, [rows, jnp.full((LANES,), L - 1, jnp.int32)], one - d_prev)

    pltpu.sync_copy(out_v, out_hbm.at[pl.ds(base, BPW)])


@functools.partial(
    pl.kernel,
    mesh=plsc.VectorSubcoreMesh(core_axis_name="c", subcore_axis_name="s"),
    out_type=jax.ShapeDtypeStruct((B, L), jnp.float32),
    compiler_params=pltpu.CompilerParams(
        needs_layout_passes=False, use_tc_tiling_on_sc=False),
    scratch_types=[
        pltpu.VMEM((BPW,), jnp.int32),
        pltpu.VMEM((BPW,), jnp.int32),
        pltpu.VMEM((BPW,), jnp.int32),
        pltpu.VMEM((BPW,), jnp.int32),
        pltpu.VMEM((BPW, D), jnp.float32),
        pltpu.VMEM((BPW, D), jnp.float32),
        pltpu.VMEM((BPW, LANES), jnp.float32),
        pltpu.VMEM((BPW, LANES), jnp.float32),
        pltpu.VMEM((BPW, L), jnp.float32),
        pltpu.SemaphoreType.DMA,
    ],
)
def _ordrec_sc(*args):
    _body(*args)


def kernel(user_ids, item_ids, user_embeddings, item_embeddings,
           item_biases, user_betas):
    ib2 = item_biases.reshape(item_biases.shape[0] // 16, 16)
    ub2 = user_betas.reshape(user_betas.shape[0] // 4, 16)
    return _ordrec_sc(user_ids.astype(jnp.int32), item_ids.astype(jnp.int32),
                      user_embeddings, item_embeddings, ib2, ub2)
